# Initial kernel scaffold; baseline (speedup 1.0000x reference)
import jax
import jax.numpy as jnp
from jax.experimental import pallas as pl


def kernel(grid_mesh_bond_embedding, grid_rect_embedding, mesh_node_embedding, G2M_edge_id2pair_tensor, G2M_edge_id_of_node_tensor, G2M_edge_coef_node_tensor, W_GM2E, g_GM2E, b_GM2E, W_E2M, g_E2M, b_E2M, W_G2G, g_G2G, b_G2G):
    raise NotImplementedError("write your pallas kernel here")


# speedup vs baseline: 1.2097x; 1.2097x over previous
import functools

import jax
import jax.numpy as jnp
from jax.experimental import pallas as pl
from jax.experimental.pallas import tpu as pltpu

D = 128


def _pick_block(n, candidates=(2048, 1280, 1024, 800, 512, 400, 256, 200, 160, 128, 80, 40, 16, 8)):
    for c in candidates:
        if n % c == 0:
            return c
    return n


def _ln(x, gamma, beta, eps=1e-5):
    mu = jnp.mean(x, axis=-1, keepdims=True)
    xc = x - mu
    var = jnp.mean(xc * xc, axis=-1, keepdims=True)
    return xc * jax.lax.rsqrt(var + eps) * gamma + beta


def _rect_body(rect_ref, wr_ref, wg_ref, g_ref, b_ref, pr_ref, outr_ref):
    x = rect_ref[...]
    pr_ref[...] = jnp.dot(x, wr_ref[...], preferred_element_type=jnp.float32)
    h = jnp.tanh(jnp.dot(x, wg_ref[...], preferred_element_type=jnp.float32))
    outr_ref[...] = x + _ln(h, g_ref[...], b_ref[...])


def _proj_body(x_ref, w_ref, out_ref):
    out_ref[...] = jnp.dot(x_ref[...], w_ref[...], preferred_element_type=jnp.float32)


def _edge_body(bond_ref, grm_ref, wb_ref, g_ref, b_ref, de_ref, ob_ref):
    x = bond_ref[...]
    h = jnp.tanh(jnp.dot(x, wb_ref[...], preferred_element_type=jnp.float32)
                 + grm_ref[...])
    d = _ln(h, g_ref[...], b_ref[...])
    de_ref[...] = d
    ob_ref[...] = x + d


def _meshout_body(mesh_ref, agg_ref, w1_ref, w2_ref, g_ref, b_ref, out_ref):
    x = mesh_ref[...]
    h = jnp.tanh(jnp.dot(x, w1_ref[...], preferred_element_type=jnp.float32)
                 + jnp.dot(agg_ref[...], w2_ref[...], preferred_element_type=jnp.float32))
    out_ref[...] = x + _ln(h, g_ref[...], b_ref[...])


def _row_spec(tile):
    return pl.BlockSpec((tile, D), lambda i: (i, 0))


def _const_spec(shape):
    return pl.BlockSpec(shape, lambda i: (0,) * len(shape))


def _rect_stage(rect, wr, wg, g, b):
    n = rect.shape[0]
    t = _pick_block(n)
    return pl.pallas_call(
        _rect_body,
        grid=(n // t,),
        in_specs=[_row_spec(t), _const_spec((D, D)), _const_spec((D, D)),
                  _const_spec((1, D)), _const_spec((1, D))],
        out_specs=[_row_spec(t), _row_spec(t)],
        out_shape=[jax.ShapeDtypeStruct((n, D), jnp.float32),
                   jax.ShapeDtypeStruct((n, D), jnp.float32)],
    )(rect, wr, wg, g, b)


def _proj_stage(x, w):
    n = x.shape[0]
    t = _pick_block(n)
    return pl.pallas_call(
        _proj_body,
        grid=(n // t,),
        in_specs=[_row_spec(t), _const_spec((D, D))],
        out_specs=_row_spec(t),
        out_shape=jax.ShapeDtypeStruct((n, D), jnp.float32),
    )(x, w)


def _edge_stage(bond, grm, wb, g, b):
    n = bond.shape[0]
    t = _pick_block(n)
    return pl.pallas_call(
        _edge_body,
        grid=(n // t,),
        in_specs=[_row_spec(t), _row_spec(t), _const_spec((D, D)),
                  _const_spec((1, D)), _const_spec((1, D))],
        out_specs=[_row_spec(t), _row_spec(t)],
        out_shape=[jax.ShapeDtypeStruct((n, D), jnp.float32),
                   jax.ShapeDtypeStruct((n, D), jnp.float32)],
    )(bond, grm, wb, g, b)


def _meshout_stage(mesh, agg, w1, w2, g, b):
    n = mesh.shape[0]
    t = _pick_block(n)
    return pl.pallas_call(
        _meshout_body,
        grid=(n // t,),
        in_specs=[_row_spec(t), _row_spec(t), _const_spec((D, D)),
                  _const_spec((D, D)), _const_spec((1, D)), _const_spec((1, D))],
        out_specs=_row_spec(t),
        out_shape=jax.ShapeDtypeStruct((n, D), jnp.float32),
    )(mesh, agg, w1, w2, g, b)


def kernel(grid_mesh_bond_embedding, grid_rect_embedding, mesh_node_embedding,
           G2M_edge_id2pair_tensor, G2M_edge_id_of_node_tensor,
           G2M_edge_coef_node_tensor,
           W_GM2E, g_GM2E, b_GM2E, W_E2M, g_E2M, b_E2M, W_G2G, g_G2G, b_G2G):
    bond = grid_mesh_bond_embedding[0]
    rect = grid_rect_embedding[0]
    mesh = mesh_node_embedding[0]
    src = G2M_edge_id2pair_tensor[:, 0]
    dst = G2M_edge_id2pair_tensor[:, 1]

    wb = W_GM2E[:, :D].T
    wr = W_GM2E[:, D:2 * D].T
    wm = W_GM2E[:, 2 * D:].T
    wm1 = W_E2M[:, :D].T
    wm2 = W_E2M[:, D:].T
    wg = W_G2G.T
    g1 = g_GM2E.reshape(1, D)
    b1 = b_GM2E.reshape(1, D)

    pr, out_rect = _rect_stage(rect, wr, wg, g_G2G.reshape(1, D), b_G2G.reshape(1, D))
    pm = _proj_stage(mesh, wm)

    grm = jnp.take(pr, src, axis=0) + jnp.take(pm, dst, axis=0)

    delta_e, out_bond = _edge_stage(bond, grm, wb, g1, b1)

    gathered = jnp.take(delta_e, G2M_edge_id_of_node_tensor.reshape(-1), axis=0)
    gathered = gathered.reshape(mesh.shape[0], -1, D)
    agg = jnp.mean(gathered * G2M_edge_coef_node_tensor, axis=-2)

    out_mesh = _meshout_stage(mesh, agg, wm1, wm2,
                              g_E2M.reshape(1, D), b_E2M.reshape(1, D))

    return (out_bond[None], out_rect[None], out_mesh[None])


# speedup vs baseline: 1.5702x; 1.2980x over previous
import functools

import jax
import jax.numpy as jnp
from jax import lax
from jax.experimental import pallas as pl
from jax.experimental.pallas import tpu as pltpu
from jax.experimental.pallas import tpu_sc as plsc

D = 128
_NC, _NS = 2, 16
_NW = _NC * _NS
_L = 16


def _pick_block(n, candidates=(2048, 1280, 1024, 800, 512, 400, 256, 200, 160, 128, 80, 40, 16, 8)):
    for c in candidates:
        if n % c == 0:
            return c
    return n


def _ln(x, gamma, beta, eps=1e-5):
    mu = jnp.mean(x, axis=-1, keepdims=True)
    xc = x - mu
    var = jnp.mean(xc * xc, axis=-1, keepdims=True)
    return xc * jax.lax.rsqrt(var + eps) * gamma + beta


def _rect_body(rect_ref, wr_ref, wg_ref, g_ref, b_ref, pr_ref, outr_ref):
    x = rect_ref[...]
    pr_ref[...] = jnp.dot(x, wr_ref[...], preferred_element_type=jnp.float32)
    h = jnp.tanh(jnp.dot(x, wg_ref[...], preferred_element_type=jnp.float32))
    outr_ref[...] = x + _ln(h, g_ref[...], b_ref[...])


def _proj_body(x_ref, w_ref, out_ref):
    out_ref[...] = jnp.dot(x_ref[...], w_ref[...], preferred_element_type=jnp.float32)


def _edge_body(bond_ref, grm_ref, wb_ref, g_ref, b_ref, de_ref, ob_ref):
    x = bond_ref[...]
    h = jnp.tanh(jnp.dot(x, wb_ref[...], preferred_element_type=jnp.float32)
                 + grm_ref[...])
    d = _ln(h, g_ref[...], b_ref[...])
    de_ref[...] = d
    ob_ref[...] = x + d


def _meshout_body(mesh_ref, agg_ref, w1_ref, w2_ref, g_ref, b_ref, out_ref):
    x = mesh_ref[...]
    h = jnp.tanh(jnp.dot(x, w1_ref[...], preferred_element_type=jnp.float32)
                 + jnp.dot(agg_ref[...], w2_ref[...], preferred_element_type=jnp.float32))
    out_ref[...] = x + _ln(h, g_ref[...], b_ref[...])


def _row_spec(tile):
    return pl.BlockSpec((tile, D), lambda i: (i, 0))


def _const_spec(shape):
    return pl.BlockSpec(shape, lambda i: (0,) * len(shape))


def _rect_stage(rect, wr, wg, g, b):
    n = rect.shape[0]
    t = _pick_block(n)
    return pl.pallas_call(
        _rect_body,
        grid=(n // t,),
        in_specs=[_row_spec(t), _const_spec((D, D)), _const_spec((D, D)),
                  _const_spec((1, D)), _const_spec((1, D))],
        out_specs=[_row_spec(t), _row_spec(t)],
        out_shape=[jax.ShapeDtypeStruct((n, D), jnp.float32),
                   jax.ShapeDtypeStruct((n, D), jnp.float32)],
    )(rect, wr, wg, g, b)


def _proj_stage(x, w):
    n = x.shape[0]
    t = _pick_block(n)
    return pl.pallas_call(
        _proj_body,
        grid=(n // t,),
        in_specs=[_row_spec(t), _const_spec((D, D))],
        out_specs=_row_spec(t),
        out_shape=jax.ShapeDtypeStruct((n, D), jnp.float32),
    )(x, w)


def _edge_stage(bond, grm, wb, g, b):
    n = bond.shape[0]
    t = _pick_block(n)
    return pl.pallas_call(
        _edge_body,
        grid=(n // t,),
        in_specs=[_row_spec(t), _row_spec(t), _const_spec((D, D)),
                  _const_spec((1, D)), _const_spec((1, D))],
        out_specs=[_row_spec(t), _row_spec(t)],
        out_shape=[jax.ShapeDtypeStruct((n, D), jnp.float32),
                   jax.ShapeDtypeStruct((n, D), jnp.float32)],
    )(bond, grm, wb, g, b)


def _meshout_stage(mesh, agg, w1, w2, g, b):
    n = mesh.shape[0]
    t = _pick_block(n)
    return pl.pallas_call(
        _meshout_body,
        grid=(n // t,),
        in_specs=[_row_spec(t), _row_spec(t), _const_spec((D, D)),
                  _const_spec((D, D)), _const_spec((1, D)), _const_spec((1, D))],
        out_specs=_row_spec(t),
        out_shape=jax.ShapeDtypeStruct((n, D), jnp.float32),
    )(mesh, agg, w1, w2, g, b)


def _sc_mesh():
    return plsc.VectorSubcoreMesh(core_axis_name="c", subcore_axis_name="s",
                                  num_cores=_NC, num_subcores=_NS)


def _edge_combine(pr, pm, src, dst):
    e = src.shape[0]
    epw = e // _NW
    ech = 8
    for cand in (128, 120, 112, 104, 96, 88, 80, 72, 64, 56, 48, 40, 32, 24, 16):
        if epw % cand == 0:
            ech = cand
            break
    nch = epw // ech
    src3 = src.reshape(_NW, nch, ech)
    dst3 = dst.reshape(_NW, nch, ech)

    @functools.partial(
        pl.kernel,
        out_type=jax.ShapeDtypeStruct((e, D), jnp.float32),
        mesh=_sc_mesh(),
        scratch_types=[
            pltpu.VMEM((nch, ech), jnp.int32),
            pltpu.VMEM((nch, ech), jnp.int32),
            pltpu.VMEM((ech, D), jnp.float32),
            pltpu.VMEM((ech, D), jnp.float32),
            pltpu.SemaphoreType.DMA,
            pltpu.SemaphoreType.DMA,
        ],
    )
    def k(pr_hbm, pm_hbm, src_hbm, dst_hbm, out_hbm,
          srcv, dstv, rows_r, rows_m, sem1, sem2):
        wid = lax.axis_index("s") * _NC + lax.axis_index("c")
        pltpu.sync_copy(src_hbm.at[wid], srcv)
        pltpu.sync_copy(dst_hbm.at[wid], dstv)

        @pl.loop(0, nch)
        def chunk(c):
            cp1 = pltpu.async_copy(pr_hbm.at[srcv.at[c]], rows_r, sem1)
            cp2 = pltpu.async_copy(pm_hbm.at[dstv.at[c]], rows_m, sem2)
            cp1.wait()
            cp2.wait()
            for r in range(ech):
                for j in range(D // _L):
                    sl = pl.ds(j * _L, _L)
                    plsc.addupdate(rows_r.at[r, sl], rows_m[r, sl])
            base = pl.multiple_of(wid * epw + c * ech, 8)
            pltpu.sync_copy(rows_r, out_hbm.at[pl.ds(base, ech)])

    return k(pr, pm, src3, dst3)


def _node_aggregate(delta_e, eid, coef, n_mesh):
    npc = 8
    k_deg = eid.shape[2] // npc
    n_pad = eid.shape[0] * eid.shape[1] * npc
    nch = eid.shape[1]
    npw = nch * npc

    @functools.partial(
        pl.kernel,
        out_type=jax.ShapeDtypeStruct((n_pad, D), jnp.float32),
        mesh=_sc_mesh(),
        scratch_types=[
            pltpu.VMEM((nch, npc * k_deg), jnp.int32),
            pltpu.VMEM((nch, npc * k_deg), jnp.float32),
            pltpu.VMEM((npc * k_deg, D), jnp.float32),
            pltpu.VMEM((npc, D), jnp.float32),
            pltpu.SemaphoreType.DMA,
        ],
    )
    def k(de_hbm, eid_hbm, coef_hbm, out_hbm, eidv, coefv, g, outb, sem):
        wid = lax.axis_index("s") * _NC + lax.axis_index("c")
        pltpu.sync_copy(eid_hbm.at[wid], eidv)
        pltpu.sync_copy(coef_hbm.at[wid], coefv)

        @pl.loop(0, nch)
        def chunk(c):
            pltpu.async_copy(de_hbm.at[eidv.at[c]], g, sem).wait()
            for j in range(npc):
                acc = [jnp.zeros((_L,), jnp.float32) for _ in range(D // _L)]
                cj = coefv[c, pl.ds(j * k_deg, k_deg)]
                for kk in range(k_deg):
                    row = j * k_deg + kk
                    s = cj[kk]
                    for dj in range(D // _L):
                        acc[dj] = acc[dj] + s * g[row, pl.ds(dj * _L, _L)]
                inv_k = jnp.float32(1.0 / k_deg)
                for dj in range(D // _L):
                    outb[j, pl.ds(dj * _L, _L)] = acc[dj] * inv_k
            base = pl.multiple_of(wid * npw + c * npc, 8)
            pltpu.sync_copy(outb, out_hbm.at[pl.ds(base, npc)])

    return k(delta_e, eid, coef)


def kernel(grid_mesh_bond_embedding, grid_rect_embedding, mesh_node_embedding,
           G2M_edge_id2pair_tensor, G2M_edge_id_of_node_tensor,
           G2M_edge_coef_node_tensor,
           W_GM2E, g_GM2E, b_GM2E, W_E2M, g_E2M, b_E2M, W_G2G, g_G2G, b_G2G):
    bond = grid_mesh_bond_embedding[0]
    rect = grid_rect_embedding[0]
    mesh = mesh_node_embedding[0]
    src = G2M_edge_id2pair_tensor[:, 0]
    dst = G2M_edge_id2pair_tensor[:, 1]

    wb = W_GM2E[:, :D].T
    wr = W_GM2E[:, D:2 * D].T
    wm = W_GM2E[:, 2 * D:].T
    wm1 = W_E2M[:, :D].T
    wm2 = W_E2M[:, D:].T
    wg = W_G2G.T
    g1 = g_GM2E.reshape(1, D)
    b1 = b_GM2E.reshape(1, D)

    pr, out_rect = _rect_stage(rect, wr, wg, g_G2G.reshape(1, D), b_G2G.reshape(1, D))
    pm = _proj_stage(mesh, wm)

    grm = _edge_combine(pr, pm, src, dst)

    delta_e, out_bond = _edge_stage(bond, grm, wb, g1, b1)

    n_mesh = mesh.shape[0]
    k_deg = G2M_edge_id_of_node_tensor.shape[1]
    npc = 8
    npw = -(-n_mesh // _NW)
    npw = -(-npw // npc) * npc
    n_pad = npw * _NW
    eid_pad = jnp.pad(G2M_edge_id_of_node_tensor, ((0, n_pad - n_mesh), (0, 0)))
    coef_pad = jnp.pad(G2M_edge_coef_node_tensor[..., 0],
                       ((0, n_pad - n_mesh), (0, 0)))
    eid3 = eid_pad.reshape(_NW, npw // npc, npc * k_deg)
    coef3 = coef_pad.reshape(_NW, npw // npc, npc * k_deg)
    agg = _node_aggregate(delta_e, eid3, coef3, n_mesh)[:n_mesh]

    out_mesh = _meshout_stage(mesh, agg, wm1, wm2,
                              g_E2M.reshape(1, D), b_E2M.reshape(1, D))

    return (out_bond[None], out_rect[None], out_mesh[None])
